# Initial kernel scaffold; baseline (speedup 1.0000x reference)
import jax
import jax.numpy as jnp
from jax.experimental import pallas as pl


def kernel(x_s, x_t, params, edge_index_s, batch_s, edge_index_t, batch_t):
    raise NotImplementedError("write your pallas kernel here")


# speedup vs baseline: 3.7566x; 3.7566x over previous
import functools

import jax
import jax.numpy as jnp
from jax import lax
from jax.experimental import pallas as pl
from jax.experimental.pallas import tpu as pltpu
from jax.experimental.pallas import tpu_sc as plsc

N = 10000
E = 320000
D = 128
C = 64
G = 64
NL = 6
CP = 2 * C

NT = 16
K = 128
BLK = 16
NBLK = 10
CH = BLK * NBLK
EPT = CH * K
NPAD = 10112
NP2 = 10240
SLAB = NP2 // NT

_mesh = plsc.VectorSubcoreMesh(core_axis_name="c", subcore_axis_name="s")


@functools.partial(
    pl.kernel,
    out_type=jax.ShapeDtypeStruct((2, NPAD), jnp.float32),
    mesh=_mesh,
    scratch_types=[
        pltpu.VMEM_SHARED((NPAD,), jnp.float32),
        pltpu.VMEM((BLK, K), jnp.int32),
        pltpu.VMEM((K,), jnp.float32),
    ],
)
def _deg_kernel(dst_hbm, zeros_hbm, out_hbm, acc, dst_blk, ones_v):
    c = lax.axis_index("c")
    s = lax.axis_index("s")
    @pl.when(s == 0)
    def _():
        pltpu.sync_copy(zeros_hbm, acc)
    for j in range(K // 16):
        ones_v[pl.ds(j * 16, 16)] = jnp.ones((16,), jnp.float32)
    plsc.subcore_barrier()

    def body(bi, carry):
        pltpu.sync_copy(dst_hbm.at[c, s, bi], dst_blk)
        for j in range(BLK):
            pltpu.sync_copy(ones_v, acc.at[dst_blk.at[j]], add=True)
        return carry

    lax.fori_loop(0, NBLK, body, 0)
    plsc.subcore_barrier()
    @pl.when(s == 0)
    def _():
        pltpu.sync_copy(acc, out_hbm.at[c])


@functools.partial(
    pl.kernel,
    out_type=jax.ShapeDtypeStruct((2, NP2, CP), jnp.float32),
    mesh=_mesh,
    scratch_types=[
        pltpu.VMEM_SHARED((NP2, CP), jnp.float32),
        pltpu.VMEM((BLK, K), jnp.int32),
        pltpu.VMEM((BLK, K), jnp.int32),
        pltpu.VMEM((K, CP), jnp.float32),
        pltpu.SemaphoreType.DMA,
    ],
)
def _scatter_kernel(a_hbm, init_hbm, src_hbm, dst_hbm, out_hbm,
                    acc, src_blk, dst_blk, rows_v, sem):
    c = lax.axis_index("c")
    s = lax.axis_index("s")
    row0 = s * SLAB
    pltpu.sync_copy(init_hbm.at[pl.ds(row0, SLAB)], acc.at[pl.ds(row0, SLAB)])
    plsc.subcore_barrier()

    def body(bi, carry):
        pltpu.sync_copy(src_hbm.at[c, s, bi], src_blk)
        pltpu.sync_copy(dst_hbm.at[c, s, bi], dst_blk)
        for j in range(BLK):
            pltpu.async_copy(a_hbm.at[src_blk.at[j]], rows_v, sem).wait()
            pltpu.sync_copy(rows_v, acc.at[dst_blk.at[j]], add=True)
        return carry

    lax.fori_loop(0, NBLK, body, 0)
    plsc.subcore_barrier()
    pltpu.sync_copy(acc.at[pl.ds(row0, SLAB)], out_hbm.at[c, pl.ds(row0, SLAB)])


def _conv_body(bn, h_ref, gamma_ref, beta_ref, w1_ref, b1_ref, w2_ref, b2_ref,
               w3_ref, b3_ref, deg_ref, a_ref, init_ref):
    if bn:
        parts = []
        for t in range(2):
            ht = h_ref[t, :N, t * C:(t + 1) * C]
            mean = jnp.mean(ht, axis=0, keepdims=True)
            xc = ht - mean
            var = jnp.mean(xc * xc, axis=0, keepdims=True)
            hn = xc * lax.rsqrt(var + 1e-5) * gamma_ref[...] + beta_ref[...]
            parts.append(jnp.maximum(hn, 0.0))
        h = jnp.concatenate(parts, axis=0)
    else:
        h = h_ref[...]
    a = jnp.dot(h, w1_ref[...], preferred_element_type=jnp.float32) + b1_ref[...]
    b = jnp.dot(h, w2_ref[...], preferred_element_type=jnp.float32) + b2_ref[...]
    cc = jnp.dot(h, w3_ref[...], preferred_element_type=jnp.float32) + b3_ref[...]
    init = cc - deg_ref[...] * b
    for t in range(2):
        a_ref[:N, t * C:(t + 1) * C] = a[t * N:(t + 1) * N]
        init_ref[:N, t * C:(t + 1) * C] = init[t * N:(t + 1) * N]


def _conv_call(bn, h, gamma, beta, w1, b1, w2, b2, w3, b3, deg):
    return pl.pallas_call(
        functools.partial(_conv_body, bn),
        out_shape=[jax.ShapeDtypeStruct((NP2, CP), jnp.float32),
                   jax.ShapeDtypeStruct((NP2, CP), jnp.float32)],
    )(h, gamma, beta, w1, b1, w2, b2, w3, b3, deg)


def _post_body(h2_ref, gamma_ref, beta_ref, batch_ref, gw1_ref, gb1_ref,
               gw2_ref, gb2_ref, ow0_ref, ob0_ref, ow1_ref, ob1_ref,
               ow2_ref, ob2_ref, ow3_ref, ob3_ref, out_ref):
    iota = lax.broadcasted_iota(jnp.int32, (N, G), 1)
    embs = []
    for t in range(2):
        ht = h2_ref[t, :N, t * C:(t + 1) * C]
        mean = jnp.mean(ht, axis=0, keepdims=True)
        xc = ht - mean
        var = jnp.mean(xc * xc, axis=0, keepdims=True)
        hn = xc * lax.rsqrt(var + 1e-5) * gamma_ref[...] + beta_ref[...]
        h = jnp.maximum(hn, 0.0)
        g1 = jnp.maximum(
            jnp.dot(h, gw1_ref[...], preferred_element_type=jnp.float32)
            + gb1_ref[...], 0.0)
        gate = jnp.maximum(
            jnp.dot(g1, gw2_ref[...], preferred_element_type=jnp.float32)
            + gb2_ref[...], 0.0)
        bt = batch_ref[t * N:(t + 1) * N]
        m = (bt == iota).astype(jnp.float32)
        masked = jnp.where(m > 0.0, gate, -1e30)
        gmax = jnp.max(masked, axis=0, keepdims=True)
        gmax = jnp.where(gmax > -1e29, gmax, 0.0)
        gmax_n = jnp.sum(m * gmax, axis=1, keepdims=True)
        e = jnp.exp(gate - gmax_n)
        esum = jnp.sum(m * e, axis=0, keepdims=True)
        esum_n = jnp.sum(m * esum, axis=1, keepdims=True)
        attn = e / (esum_n + 1e-16)
        emb = lax.dot_general(m, attn * h, (((0,), (0,)), ((), ())),
                              preferred_element_type=jnp.float32)
        embs.append(emb)
    hm = jnp.abs(embs[0] - embs[1])
    for w_ref, b_ref in ((ow0_ref, ob0_ref), (ow1_ref, ob1_ref),
                         (ow2_ref, ob2_ref)):
        hm = jnp.maximum(
            jnp.dot(hm, w_ref[...], preferred_element_type=jnp.float32)
            + b_ref[...], 0.0)
    out_ref[...] = (jnp.dot(hm, ow3_ref[...], preferred_element_type=jnp.float32)
                    + ob3_ref[...])


def kernel(x_s, x_t, params, edge_index_s, batch_s, edge_index_t, batch_t):
    f32 = jnp.float32
    x = jnp.concatenate([x_s, x_t], axis=0)
    batch = jnp.concatenate([batch_s, batch_t])[:, None]

    npad_e = NT * EPT - E
    def prep(ei):
        src = jnp.concatenate([ei[0], jnp.zeros((npad_e,), jnp.int32)])
        dst = jnp.concatenate([ei[1], jnp.full((npad_e,), N, jnp.int32)])
        return (src.reshape(NT, NBLK, BLK, K), dst.reshape(NT, NBLK, BLK, K))
    src_s4, dst_s4 = prep(edge_index_s)
    src_t4, dst_t4 = prep(edge_index_t)
    src4 = jnp.stack([src_s4, src_t4])
    dst4 = jnp.stack([dst_s4, dst_t4])

    deg2 = _deg_kernel(dst4, jnp.zeros((NPAD,), f32))
    deg = deg2[:, :N].reshape(2 * N, 1)

    def p(name):
        return params[name]

    def row(v):
        return v.reshape(1, -1)

    h = x
    for l in range(NL):
        bn = l > 0
        gamma = row(p("conv%d_gamma" % (l - 1))) if bn else row(jnp.ones((C,), f32))
        beta = row(p("conv%d_beta" % (l - 1))) if bn else row(jnp.zeros((C,), f32))
        a, init = _conv_call(
            bn, h, gamma, beta,
            p("conv%d_W1" % l), row(p("conv%d_b1" % l)),
            p("conv%d_W2" % l), row(p("conv%d_b2" % l)),
            p("conv%d_W3" % l), row(p("conv%d_b3" % l)), deg)
        h = _scatter_kernel(a, init, src4, dst4)

    out = pl.pallas_call(
        _post_body,
        out_shape=jax.ShapeDtypeStruct((G, 1), jnp.float32),
    )(h, row(p("conv%d_gamma" % (NL - 1))), row(p("conv%d_beta" % (NL - 1))),
      batch,
      p("gate_W1"), row(p("gate_b1")), p("gate_W2"), row(p("gate_b2")),
      p("out_W0"), row(p("out_b0")), p("out_W1"), row(p("out_b1")),
      p("out_W2"), row(p("out_b2")), p("out_W3"), row(p("out_b3")))
    return out


# speedup vs baseline: 3.9925x; 1.0628x over previous
import functools

import jax
import jax.numpy as jnp
from jax import lax
from jax.experimental import pallas as pl
from jax.experimental.pallas import tpu as pltpu
from jax.experimental.pallas import tpu_sc as plsc

N = 10000
E = 320000
D = 128
C = 64
G = 64
NL = 6
CP = 2 * C

NT = 16
K = 128
BLK = 16
NBLK = 10
CH = BLK * NBLK
EPT = CH * K
NPAD = 10112
NP2 = 10240
SLAB = NP2 // NT

_mesh = plsc.VectorSubcoreMesh(core_axis_name="c", subcore_axis_name="s")


@functools.partial(
    pl.kernel,
    out_type=jax.ShapeDtypeStruct((2, NPAD), jnp.float32),
    mesh=_mesh,
    scratch_types=[
        pltpu.VMEM_SHARED((NPAD,), jnp.float32),
        pltpu.VMEM((BLK, K), jnp.int32),
        pltpu.VMEM((K,), jnp.float32),
    ],
)
def _deg_kernel(dst_hbm, zeros_hbm, out_hbm, acc, dst_blk, ones_v):
    c = lax.axis_index("c")
    s = lax.axis_index("s")
    @pl.when(s == 0)
    def _():
        pltpu.sync_copy(zeros_hbm, acc)
    for j in range(K // 16):
        ones_v[pl.ds(j * 16, 16)] = jnp.ones((16,), jnp.float32)
    plsc.subcore_barrier()

    def body(bi, carry):
        pltpu.sync_copy(dst_hbm.at[c, s, bi], dst_blk)
        for j in range(BLK):
            pltpu.sync_copy(ones_v, acc.at[dst_blk.at[j]], add=True)
        return carry

    lax.fori_loop(0, NBLK, body, 0)
    plsc.subcore_barrier()
    @pl.when(s == 0)
    def _():
        pltpu.sync_copy(acc, out_hbm.at[c])


@functools.partial(
    pl.kernel,
    out_type=jax.ShapeDtypeStruct((2, NP2, CP), jnp.float32),
    mesh=_mesh,
    scratch_types=[
        pltpu.VMEM_SHARED((NP2, CP), jnp.float32),
        pltpu.VMEM((BLK, K), jnp.int32),
        pltpu.VMEM((BLK, K), jnp.int32),
        pltpu.VMEM((K, CP), jnp.float32),
        pltpu.VMEM((K, CP), jnp.float32),
        pltpu.SemaphoreType.DMA,
        pltpu.SemaphoreType.DMA,
    ],
)
def _scatter_kernel(a_hbm, init_hbm, src_hbm, dst_hbm, out_hbm,
                    acc, src_blk, dst_blk, rows0, rows1, semg, sems):
    c = lax.axis_index("c")
    s = lax.axis_index("s")
    row0 = s * SLAB
    pltpu.sync_copy(init_hbm.at[pl.ds(row0, SLAB)], acc.at[pl.ds(row0, SLAB)])
    plsc.subcore_barrier()
    rows = (rows0, rows1)

    def body(bi, carry):
        pltpu.sync_copy(src_hbm.at[c, s, bi], src_blk)
        pltpu.sync_copy(dst_hbm.at[c, s, bi], dst_blk)
        g = pltpu.async_copy(a_hbm.at[src_blk.at[0]], rows[0], semg)
        prev_sc = None
        for j in range(BLK):
            g.wait()
            sc = pltpu.async_copy(rows[j % 2], acc.at[dst_blk.at[j]], sems,
                                  add=True)
            if prev_sc is not None:
                prev_sc.wait()
            if j + 1 < BLK:
                g = pltpu.async_copy(a_hbm.at[src_blk.at[j + 1]],
                                     rows[(j + 1) % 2], semg)
            prev_sc = sc
        prev_sc.wait()
        return carry

    lax.fori_loop(0, NBLK, body, 0)
    plsc.subcore_barrier()
    pltpu.sync_copy(acc.at[pl.ds(row0, SLAB)], out_hbm.at[c, pl.ds(row0, SLAB)])


def _conv_body(bn, h_ref, gamma_ref, beta_ref, w1_ref, b1_ref, w2_ref, b2_ref,
               w3_ref, b3_ref, deg_ref, a_ref, init_ref):
    if bn:
        parts = []
        for t in range(2):
            ht = h_ref[t, :N, t * C:(t + 1) * C]
            mean = jnp.mean(ht, axis=0, keepdims=True)
            xc = ht - mean
            var = jnp.mean(xc * xc, axis=0, keepdims=True)
            hn = xc * lax.rsqrt(var + 1e-5) * gamma_ref[...] + beta_ref[...]
            parts.append(jnp.maximum(hn, 0.0))
        h = jnp.concatenate(parts, axis=0)
    else:
        h = h_ref[...]
    a = jnp.dot(h, w1_ref[...], preferred_element_type=jnp.float32) + b1_ref[...]
    b = jnp.dot(h, w2_ref[...], preferred_element_type=jnp.float32) + b2_ref[...]
    cc = jnp.dot(h, w3_ref[...], preferred_element_type=jnp.float32) + b3_ref[...]
    init = cc - deg_ref[...] * b
    for t in range(2):
        a_ref[:N, t * C:(t + 1) * C] = a[t * N:(t + 1) * N]
        init_ref[:N, t * C:(t + 1) * C] = init[t * N:(t + 1) * N]


def _conv_call(bn, h, gamma, beta, w1, b1, w2, b2, w3, b3, deg):
    return pl.pallas_call(
        functools.partial(_conv_body, bn),
        out_shape=[jax.ShapeDtypeStruct((NP2, CP), jnp.float32),
                   jax.ShapeDtypeStruct((NP2, CP), jnp.float32)],
    )(h, gamma, beta, w1, b1, w2, b2, w3, b3, deg)


def _post_body(h2_ref, gamma_ref, beta_ref, batch_ref, gw1_ref, gb1_ref,
               gw2_ref, gb2_ref, ow0_ref, ob0_ref, ow1_ref, ob1_ref,
               ow2_ref, ob2_ref, ow3_ref, ob3_ref, out_ref):
    iota = lax.broadcasted_iota(jnp.int32, (N, G), 1)
    embs = []
    for t in range(2):
        ht = h2_ref[t, :N, t * C:(t + 1) * C]
        mean = jnp.mean(ht, axis=0, keepdims=True)
        xc = ht - mean
        var = jnp.mean(xc * xc, axis=0, keepdims=True)
        hn = xc * lax.rsqrt(var + 1e-5) * gamma_ref[...] + beta_ref[...]
        h = jnp.maximum(hn, 0.0)
        g1 = jnp.maximum(
            jnp.dot(h, gw1_ref[...], preferred_element_type=jnp.float32)
            + gb1_ref[...], 0.0)
        gate = jnp.maximum(
            jnp.dot(g1, gw2_ref[...], preferred_element_type=jnp.float32)
            + gb2_ref[...], 0.0)
        bt = batch_ref[t * N:(t + 1) * N]
        m = (bt == iota).astype(jnp.float32)
        masked = jnp.where(m > 0.0, gate, -1e30)
        gmax = jnp.max(masked, axis=0, keepdims=True)
        gmax = jnp.where(gmax > -1e29, gmax, 0.0)
        gmax_n = jnp.sum(m * gmax, axis=1, keepdims=True)
        e = jnp.exp(gate - gmax_n)
        esum = jnp.sum(m * e, axis=0, keepdims=True)
        esum_n = jnp.sum(m * esum, axis=1, keepdims=True)
        attn = e / (esum_n + 1e-16)
        emb = lax.dot_general(m, attn * h, (((0,), (0,)), ((), ())),
                              preferred_element_type=jnp.float32)
        embs.append(emb)
    hm = jnp.abs(embs[0] - embs[1])
    for w_ref, b_ref in ((ow0_ref, ob0_ref), (ow1_ref, ob1_ref),
                         (ow2_ref, ob2_ref)):
        hm = jnp.maximum(
            jnp.dot(hm, w_ref[...], preferred_element_type=jnp.float32)
            + b_ref[...], 0.0)
    out_ref[...] = (jnp.dot(hm, ow3_ref[...], preferred_element_type=jnp.float32)
                    + ob3_ref[...])


def kernel(x_s, x_t, params, edge_index_s, batch_s, edge_index_t, batch_t):
    f32 = jnp.float32
    x = jnp.concatenate([x_s, x_t], axis=0)
    batch = jnp.concatenate([batch_s, batch_t])[:, None]

    npad_e = NT * EPT - E
    def prep(ei):
        src = jnp.concatenate([ei[0], jnp.zeros((npad_e,), jnp.int32)])
        dst = jnp.concatenate([ei[1], jnp.full((npad_e,), N, jnp.int32)])
        return (src.reshape(NT, NBLK, BLK, K), dst.reshape(NT, NBLK, BLK, K))
    src_s4, dst_s4 = prep(edge_index_s)
    src_t4, dst_t4 = prep(edge_index_t)
    src4 = jnp.stack([src_s4, src_t4])
    dst4 = jnp.stack([dst_s4, dst_t4])

    deg2 = _deg_kernel(dst4, jnp.zeros((NPAD,), f32))
    deg = deg2[:, :N].reshape(2 * N, 1)

    def p(name):
        return params[name]

    def row(v):
        return v.reshape(1, -1)

    h = x
    for l in range(NL):
        bn = l > 0
        gamma = row(p("conv%d_gamma" % (l - 1))) if bn else row(jnp.ones((C,), f32))
        beta = row(p("conv%d_beta" % (l - 1))) if bn else row(jnp.zeros((C,), f32))
        a, init = _conv_call(
            bn, h, gamma, beta,
            p("conv%d_W1" % l), row(p("conv%d_b1" % l)),
            p("conv%d_W2" % l), row(p("conv%d_b2" % l)),
            p("conv%d_W3" % l), row(p("conv%d_b3" % l)), deg)
        h = _scatter_kernel(a, init, src4, dst4)

    out = pl.pallas_call(
        _post_body,
        out_shape=jax.ShapeDtypeStruct((G, 1), jnp.float32),
    )(h, row(p("conv%d_gamma" % (NL - 1))), row(p("conv%d_beta" % (NL - 1))),
      batch,
      p("gate_W1"), row(p("gate_b1")), p("gate_W2"), row(p("gate_b2")),
      p("out_W0"), row(p("out_b0")), p("out_W1"), row(p("out_b1")),
      p("out_W2"), row(p("out_b2")), p("out_W3"), row(p("out_b3")))
    return out


# speedup vs baseline: 15.8851x; 3.9787x over previous
import functools

import jax
import jax.numpy as jnp
from jax import lax
from jax.experimental import pallas as pl
from jax.experimental.pallas import tpu as pltpu
from jax.experimental.pallas import tpu_sc as plsc

N = 10000
E = 320000
D = 128
C = 64
G = 64
NL = 6

NT = 16
K = 128
BLK = 16
NBLK = 10
CH = BLK * NBLK
EPT = CH * K
NPAD = 10112
NP2 = 10240
SLAB = NP2 // NT

_mesh = plsc.VectorSubcoreMesh(core_axis_name="c", subcore_axis_name="s")


@functools.partial(
    pl.kernel,
    out_type=jax.ShapeDtypeStruct((2, NPAD), jnp.float32),
    mesh=_mesh,
    scratch_types=[
        pltpu.VMEM_SHARED((NPAD,), jnp.float32),
        pltpu.VMEM((BLK, K), jnp.int32),
        pltpu.VMEM((K,), jnp.float32),
    ],
)
def _deg_kernel(dst_hbm, zeros_hbm, out_hbm, acc, dst_blk, ones_v):
    c = lax.axis_index("c")
    s = lax.axis_index("s")
    @pl.when(s == 0)
    def _():
        pltpu.sync_copy(zeros_hbm, acc)
    for j in range(K // 16):
        ones_v[pl.ds(j * 16, 16)] = jnp.ones((16,), jnp.float32)
    plsc.subcore_barrier()

    def body(bi, carry):
        pltpu.sync_copy(dst_hbm.at[c, s, bi], dst_blk)
        for j in range(BLK):
            pltpu.sync_copy(ones_v, acc.at[dst_blk.at[j]], add=True)
        return carry

    lax.fori_loop(0, NBLK, body, 0)
    plsc.subcore_barrier()
    @pl.when(s == 0)
    def _():
        pltpu.sync_copy(acc, out_hbm.at[c])


@functools.partial(
    pl.kernel,
    out_type=jax.ShapeDtypeStruct((2, NP2, C), jnp.float32),
    mesh=_mesh,
    scratch_types=[
        pltpu.VMEM_SHARED((NP2, C), jnp.float32),
        pltpu.VMEM_SHARED((NP2, C), jnp.float32),
        pltpu.VMEM((BLK, K), jnp.int32),
        pltpu.VMEM((BLK, K), jnp.int32),
        pltpu.VMEM((K, C), jnp.float32),
        pltpu.VMEM((K, C), jnp.float32),
        pltpu.SemaphoreType.DMA,
        pltpu.SemaphoreType.DMA,
    ],
)
def _scatter_kernel(a_hbm, init_hbm, src_hbm, dst_hbm, out_hbm,
                    a_sp, acc, src_blk, dst_blk, rows0, rows1, semg, sems):
    c = lax.axis_index("c")
    s = lax.axis_index("s")
    row0 = s * SLAB
    pltpu.sync_copy(a_hbm.at[c, pl.ds(row0, SLAB)], a_sp.at[pl.ds(row0, SLAB)])
    pltpu.sync_copy(init_hbm.at[c, pl.ds(row0, SLAB)], acc.at[pl.ds(row0, SLAB)])
    plsc.subcore_barrier()
    rows = (rows0, rows1)

    def body(bi, carry):
        pltpu.sync_copy(src_hbm.at[c, s, bi], src_blk)
        pltpu.sync_copy(dst_hbm.at[c, s, bi], dst_blk)
        g = pltpu.async_copy(a_sp.at[src_blk.at[0]], rows[0], semg)
        prev_sc = None
        for j in range(BLK):
            g.wait()
            sc = pltpu.async_copy(rows[j % 2], acc.at[dst_blk.at[j]], sems,
                                  add=True)
            if prev_sc is not None:
                prev_sc.wait()
            if j + 1 < BLK:
                g = pltpu.async_copy(a_sp.at[src_blk.at[j + 1]],
                                     rows[(j + 1) % 2], semg)
            prev_sc = sc
        prev_sc.wait()
        return carry

    lax.fori_loop(0, NBLK, body, 0)
    plsc.subcore_barrier()
    pltpu.sync_copy(acc.at[pl.ds(row0, SLAB)], out_hbm.at[c, pl.ds(row0, SLAB)])


RB = 2000
NB = N // RB


def _stats_body(h2_ref, gamma_ref, beta_ref, stats_ref):
    for t in range(2):
        ht = h2_ref[t, :N, :]
        mean = jnp.mean(ht, axis=0, keepdims=True)
        xc = ht - mean
        var = jnp.mean(xc * xc, axis=0, keepdims=True)
        scale = lax.rsqrt(var + 1e-5) * gamma_ref[...]
        stats_ref[t, 0:1, :] = scale
        stats_ref[t, 1:2, :] = beta_ref[...] - mean * scale


def _stats_call(h2, gamma, beta):
    return pl.pallas_call(
        _stats_body,
        out_shape=jax.ShapeDtypeStruct((2, 8, C), jnp.float32),
    )(h2, gamma, beta)


def _conv_body(bn, h_ref, stats_ref, w1_ref, b1_ref, w2_ref, b2_ref,
               w3_ref, b3_ref, deg_ref, a_ref, init_ref):
    if bn:
        hb = h_ref[0]
        scale = stats_ref[0, 0:1, :]
        shift = stats_ref[0, 1:2, :]
        h = jnp.maximum(hb * scale + shift, 0.0)
    else:
        h = h_ref[...]
    a = jnp.dot(h, w1_ref[...], preferred_element_type=jnp.float32) + b1_ref[...]
    b = jnp.dot(h, w2_ref[...], preferred_element_type=jnp.float32) + b2_ref[...]
    cc = jnp.dot(h, w3_ref[...], preferred_element_type=jnp.float32) + b3_ref[...]
    a_ref[0] = a
    init_ref[0] = cc - deg_ref[0] * b


def _conv_call(bn, h, stats, w1, b1, w2, b2, w3, b3, deg3):
    full = lambda arr: pl.BlockSpec(arr.shape, lambda t, i: (0,) * arr.ndim)
    if bn:
        h_spec = pl.BlockSpec((1, RB, C), lambda t, i: (t, i, 0))
    else:
        h_spec = pl.BlockSpec((RB, D), lambda t, i: (t * NB + i, 0))
    return pl.pallas_call(
        functools.partial(_conv_body, bn),
        grid=(2, NB),
        in_specs=[h_spec,
                  pl.BlockSpec((1, 8, C), lambda t, i: (t, 0, 0)),
                  full(w1), full(b1), full(w2), full(b2), full(w3), full(b3),
                  pl.BlockSpec((1, RB, 1), lambda t, i: (t, i, 0))],
        out_specs=[pl.BlockSpec((1, RB, C), lambda t, i: (t, i, 0)),
                   pl.BlockSpec((1, RB, C), lambda t, i: (t, i, 0))],
        out_shape=[jax.ShapeDtypeStruct((2, NP2, C), jnp.float32),
                   jax.ShapeDtypeStruct((2, NP2, C), jnp.float32)],
    )(h, stats, w1, b1, w2, b2, w3, b3, deg3)


def _post_body(h2_ref, gamma_ref, beta_ref, batch_ref, gw1_ref, gb1_ref,
               gw2_ref, gb2_ref, ow0_ref, ob0_ref, ow1_ref, ob1_ref,
               ow2_ref, ob2_ref, ow3_ref, ob3_ref, out_ref):
    iota = lax.broadcasted_iota(jnp.int32, (N, G), 1)
    embs = []
    for t in range(2):
        ht = h2_ref[t, :N, :]
        mean = jnp.mean(ht, axis=0, keepdims=True)
        xc = ht - mean
        var = jnp.mean(xc * xc, axis=0, keepdims=True)
        hn = xc * lax.rsqrt(var + 1e-5) * gamma_ref[...] + beta_ref[...]
        h = jnp.maximum(hn, 0.0)
        g1 = jnp.maximum(
            jnp.dot(h, gw1_ref[...], preferred_element_type=jnp.float32)
            + gb1_ref[...], 0.0)
        gate = jnp.maximum(
            jnp.dot(g1, gw2_ref[...], preferred_element_type=jnp.float32)
            + gb2_ref[...], 0.0)
        bt = batch_ref[t * N:(t + 1) * N]
        m = (bt == iota).astype(jnp.float32)
        masked = jnp.where(m > 0.0, gate, -1e30)
        gmax = jnp.max(masked, axis=0, keepdims=True)
        gmax = jnp.where(gmax > -1e29, gmax, 0.0)
        gmax_n = jnp.sum(m * gmax, axis=1, keepdims=True)
        e = jnp.exp(gate - gmax_n)
        esum = jnp.sum(m * e, axis=0, keepdims=True)
        esum_n = jnp.sum(m * esum, axis=1, keepdims=True)
        attn = e / (esum_n + 1e-16)
        emb = lax.dot_general(m, attn * h, (((0,), (0,)), ((), ())),
                              preferred_element_type=jnp.float32)
        embs.append(emb)
    hm = jnp.abs(embs[0] - embs[1])
    for w_ref, b_ref in ((ow0_ref, ob0_ref), (ow1_ref, ob1_ref),
                         (ow2_ref, ob2_ref)):
        hm = jnp.maximum(
            jnp.dot(hm, w_ref[...], preferred_element_type=jnp.float32)
            + b_ref[...], 0.0)
    out_ref[...] = (jnp.dot(hm, ow3_ref[...], preferred_element_type=jnp.float32)
                    + ob3_ref[...])


def kernel(x_s, x_t, params, edge_index_s, batch_s, edge_index_t, batch_t):
    f32 = jnp.float32
    x = jnp.concatenate([x_s, x_t], axis=0)
    batch = jnp.concatenate([batch_s, batch_t])[:, None]

    npad_e = NT * EPT - E
    def prep(ei):
        src = jnp.concatenate([ei[0], jnp.zeros((npad_e,), jnp.int32)])
        dst = jnp.concatenate([ei[1], jnp.full((npad_e,), N, jnp.int32)])
        return (src.reshape(NT, NBLK, BLK, K), dst.reshape(NT, NBLK, BLK, K))
    src_s4, dst_s4 = prep(edge_index_s)
    src_t4, dst_t4 = prep(edge_index_t)
    src4 = jnp.stack([src_s4, src_t4])
    dst4 = jnp.stack([dst_s4, dst_t4])

    deg2 = _deg_kernel(dst4, jnp.zeros((NPAD,), f32))
    deg3 = jnp.concatenate(
        [deg2[:, :N], jnp.zeros((2, NP2 - N), f32)], axis=1)[..., None]

    def p(name):
        return params[name]

    def row(v):
        return v.reshape(1, -1)

    h = x
    for l in range(NL):
        bn = l > 0
        if bn:
            stats = _stats_call(h, row(p("conv%d_gamma" % (l - 1))),
                                row(p("conv%d_beta" % (l - 1))))
        else:
            stats = jnp.zeros((2, 8, C), f32)
        a, init = _conv_call(
            bn, h, stats,
            p("conv%d_W1" % l), row(p("conv%d_b1" % l)),
            p("conv%d_W2" % l), row(p("conv%d_b2" % l)),
            p("conv%d_W3" % l), row(p("conv%d_b3" % l)), deg3)
        h = _scatter_kernel(a, init, src4, dst4)

    out = pl.pallas_call(
        _post_body,
        out_shape=jax.ShapeDtypeStruct((G, 1), jnp.float32),
    )(h, row(p("conv%d_gamma" % (NL - 1))), row(p("conv%d_beta" % (NL - 1))),
      batch,
      p("gate_W1"), row(p("gate_b1")), p("gate_W2"), row(p("gate_b2")),
      p("out_W0"), row(p("out_b0")), p("out_W1"), row(p("out_b1")),
      p("out_W2"), row(p("out_b2")), p("out_W3"), row(p("out_b3")))
    return out


# speedup vs baseline: 16.1249x; 1.0151x over previous
import functools

import jax
import jax.numpy as jnp
from jax import lax
from jax.experimental import pallas as pl
from jax.experimental.pallas import tpu as pltpu
from jax.experimental.pallas import tpu_sc as plsc

N = 10000
E = 320000
D = 128
C = 64
G = 64
NL = 6

NT = 16
K = 128
BLK = 32
NBLK = 5
CH = BLK * NBLK
EPT = CH * K
NPAD = 10112
NP2 = 10240
SLAB = NP2 // NT

_mesh = plsc.VectorSubcoreMesh(core_axis_name="c", subcore_axis_name="s")


@functools.partial(
    pl.kernel,
    out_type=jax.ShapeDtypeStruct((2, NPAD), jnp.float32),
    mesh=_mesh,
    scratch_types=[
        pltpu.VMEM_SHARED((NPAD,), jnp.float32),
        pltpu.VMEM((BLK, K), jnp.int32),
        pltpu.VMEM((K,), jnp.float32),
    ],
)
def _deg_kernel(dst_hbm, zeros_hbm, out_hbm, acc, dst_blk, ones_v):
    c = lax.axis_index("c")
    s = lax.axis_index("s")
    @pl.when(s == 0)
    def _():
        pltpu.sync_copy(zeros_hbm, acc)
    for j in range(K // 16):
        ones_v[pl.ds(j * 16, 16)] = jnp.ones((16,), jnp.float32)
    plsc.subcore_barrier()

    def body(bi, carry):
        pltpu.sync_copy(dst_hbm.at[c, s, bi], dst_blk)
        for j in range(BLK):
            pltpu.sync_copy(ones_v, acc.at[dst_blk.at[j]], add=True)
        return carry

    lax.fori_loop(0, NBLK, body, 0)
    plsc.subcore_barrier()
    @pl.when(s == 0)
    def _():
        pltpu.sync_copy(acc, out_hbm.at[c])


@functools.partial(
    pl.kernel,
    out_type=jax.ShapeDtypeStruct((2, NP2, C), jnp.float32),
    mesh=_mesh,
    scratch_types=[
        pltpu.VMEM_SHARED((NP2, C), jnp.float32),
        pltpu.VMEM_SHARED((NP2, C), jnp.float32),
        pltpu.VMEM((BLK, K), jnp.int32),
        pltpu.VMEM((BLK, K), jnp.int32),
        pltpu.VMEM((K, C), jnp.float32),
        pltpu.VMEM((K, C), jnp.float32),
        pltpu.SemaphoreType.DMA,
        pltpu.SemaphoreType.DMA,
    ],
)
def _scatter_kernel(a_hbm, init_hbm, src_hbm, dst_hbm, out_hbm,
                    a_sp, acc, src_blk, dst_blk, rows0, rows1, semg, sems):
    c = lax.axis_index("c")
    s = lax.axis_index("s")
    row0 = s * SLAB
    pltpu.sync_copy(a_hbm.at[c, pl.ds(row0, SLAB)], a_sp.at[pl.ds(row0, SLAB)])
    pltpu.sync_copy(init_hbm.at[c, pl.ds(row0, SLAB)], acc.at[pl.ds(row0, SLAB)])
    plsc.subcore_barrier()
    rows = (rows0, rows1)

    def body(bi, carry):
        pltpu.sync_copy(src_hbm.at[c, s, bi], src_blk)
        pltpu.sync_copy(dst_hbm.at[c, s, bi], dst_blk)
        g = pltpu.async_copy(a_sp.at[src_blk.at[0]], rows[0], semg)
        prev_sc = None
        for j in range(BLK):
            g.wait()
            sc = pltpu.async_copy(rows[j % 2], acc.at[dst_blk.at[j]], sems,
                                  add=True)
            if prev_sc is not None:
                prev_sc.wait()
            if j + 1 < BLK:
                g = pltpu.async_copy(a_sp.at[src_blk.at[j + 1]],
                                     rows[(j + 1) % 2], semg)
            prev_sc = sc
        prev_sc.wait()
        return carry

    lax.fori_loop(0, NBLK, body, 0)
    plsc.subcore_barrier()
    pltpu.sync_copy(acc.at[pl.ds(row0, SLAB)], out_hbm.at[c, pl.ds(row0, SLAB)])


RB = 2000
NB = N // RB


def _mm_tail(h, w1_ref, b1_ref, w2_ref, b2_ref, w3_ref, b3_ref, degb,
             a_ref, init_ref):
    a = jnp.dot(h, w1_ref[...], preferred_element_type=jnp.float32) + b1_ref[...]
    b = jnp.dot(h, w2_ref[...], preferred_element_type=jnp.float32) + b2_ref[...]
    cc = jnp.dot(h, w3_ref[...], preferred_element_type=jnp.float32) + b3_ref[...]
    a_ref[0] = a
    init_ref[0] = cc - degb * b


def _conv0_body(h_ref, w1_ref, b1_ref, w2_ref, b2_ref, w3_ref, b3_ref,
                deg_ref, a_ref, init_ref):
    _mm_tail(h_ref[...], w1_ref, b1_ref, w2_ref, b2_ref, w3_ref, b3_ref,
             deg_ref[0], a_ref, init_ref)


def _conv0_call(x, w1, b1, w2, b2, w3, b3, deg3):
    full = lambda arr: pl.BlockSpec(arr.shape, lambda t, i: (0,) * arr.ndim)
    return pl.pallas_call(
        _conv0_body,
        grid=(2, NB),
        in_specs=[pl.BlockSpec((RB, D), lambda t, i: (t * NB + i, 0)),
                  full(w1), full(b1), full(w2), full(b2), full(w3), full(b3),
                  pl.BlockSpec((1, RB, 1), lambda t, i: (t, i, 0))],
        out_specs=[pl.BlockSpec((1, RB, C), lambda t, i: (t, i, 0)),
                   pl.BlockSpec((1, RB, C), lambda t, i: (t, i, 0))],
        out_shape=[jax.ShapeDtypeStruct((2, NP2, C), jnp.float32),
                   jax.ShapeDtypeStruct((2, NP2, C), jnp.float32)],
    )(x, w1, b1, w2, b2, w3, b3, deg3)


def _convbn_body(h_ref, gamma_ref, beta_ref, w1_ref, b1_ref, w2_ref, b2_ref,
                 w3_ref, b3_ref, deg_ref, a_ref, init_ref, sums):
    i = pl.program_id(1)
    phase0 = i < NB

    @pl.when(i == 0)
    def _():
        sums[0:2, :] = jnp.zeros((2, C), jnp.float32)

    @pl.when(phase0)
    def _():
        hb = h_ref[0]
        sums[0:1, :] += jnp.sum(hb, axis=0, keepdims=True)
        sums[1:2, :] += jnp.sum(hb * hb, axis=0, keepdims=True)

    @pl.when(i == NB - 1)
    def _():
        mean = sums[0:1, :] * (1.0 / N)
        var = sums[1:2, :] * (1.0 / N) - mean * mean
        scale = lax.rsqrt(var + 1e-5) * gamma_ref[...]
        sums[2:3, :] = scale
        sums[3:4, :] = beta_ref[...] - mean * scale

    @pl.when(jnp.logical_not(phase0))
    def _():
        h = jnp.maximum(h_ref[0] * sums[2:3, :] + sums[3:4, :], 0.0)
        _mm_tail(h, w1_ref, b1_ref, w2_ref, b2_ref, w3_ref, b3_ref,
                 deg_ref[0], a_ref, init_ref)


def _convbn_call(h, gamma, beta, w1, b1, w2, b2, w3, b3, deg3):
    full = lambda arr: pl.BlockSpec(arr.shape, lambda t, i: (0,) * arr.ndim)
    blk = lambda t, i: (t, lax.rem(i, NB), 0)
    return pl.pallas_call(
        _convbn_body,
        grid=(2, 2 * NB),
        in_specs=[pl.BlockSpec((1, RB, C), blk),
                  full(gamma), full(beta),
                  full(w1), full(b1), full(w2), full(b2), full(w3), full(b3),
                  pl.BlockSpec((1, RB, 1), blk)],
        out_specs=[pl.BlockSpec((1, RB, C), blk),
                   pl.BlockSpec((1, RB, C), blk)],
        out_shape=[jax.ShapeDtypeStruct((2, NP2, C), jnp.float32),
                   jax.ShapeDtypeStruct((2, NP2, C), jnp.float32)],
        scratch_shapes=[pltpu.VMEM((8, C), jnp.float32)],
    )(h, gamma, beta, w1, b1, w2, b2, w3, b3, deg3)


def _post_body(h2_ref, gamma_ref, beta_ref, batch_ref, gw1_ref, gb1_ref,
               gw2_ref, gb2_ref, ow0_ref, ob0_ref, ow1_ref, ob1_ref,
               ow2_ref, ob2_ref, ow3_ref, ob3_ref, out_ref):
    iota = lax.broadcasted_iota(jnp.int32, (N, G), 1)
    embs = []
    for t in range(2):
        ht = h2_ref[t, :N, :]
        mean = jnp.mean(ht, axis=0, keepdims=True)
        xc = ht - mean
        var = jnp.mean(xc * xc, axis=0, keepdims=True)
        hn = xc * lax.rsqrt(var + 1e-5) * gamma_ref[...] + beta_ref[...]
        h = jnp.maximum(hn, 0.0)
        g1 = jnp.maximum(
            jnp.dot(h, gw1_ref[...], preferred_element_type=jnp.float32)
            + gb1_ref[...], 0.0)
        gate = jnp.maximum(
            jnp.dot(g1, gw2_ref[...], preferred_element_type=jnp.float32)
            + gb2_ref[...], 0.0)
        bt = batch_ref[t * N:(t + 1) * N]
        m = (bt == iota).astype(jnp.float32)
        masked = jnp.where(m > 0.0, gate, -1e30)
        gmax = jnp.max(masked, axis=0, keepdims=True)
        gmax = jnp.where(gmax > -1e29, gmax, 0.0)
        gmax_n = jnp.sum(m * gmax, axis=1, keepdims=True)
        e = jnp.exp(gate - gmax_n)
        esum = jnp.sum(m * e, axis=0, keepdims=True)
        esum_n = jnp.sum(m * esum, axis=1, keepdims=True)
        attn = e / (esum_n + 1e-16)
        emb = lax.dot_general(m, attn * h, (((0,), (0,)), ((), ())),
                              preferred_element_type=jnp.float32)
        embs.append(emb)
    hm = jnp.abs(embs[0] - embs[1])
    for w_ref, b_ref in ((ow0_ref, ob0_ref), (ow1_ref, ob1_ref),
                         (ow2_ref, ob2_ref)):
        hm = jnp.maximum(
            jnp.dot(hm, w_ref[...], preferred_element_type=jnp.float32)
            + b_ref[...], 0.0)
    out_ref[...] = (jnp.dot(hm, ow3_ref[...], preferred_element_type=jnp.float32)
                    + ob3_ref[...])


def kernel(x_s, x_t, params, edge_index_s, batch_s, edge_index_t, batch_t):
    f32 = jnp.float32
    x = jnp.concatenate([x_s, x_t], axis=0)
    batch = jnp.concatenate([batch_s, batch_t])[:, None]

    npad_e = NT * EPT - E
    def prep(ei):
        src = jnp.concatenate([ei[0], jnp.zeros((npad_e,), jnp.int32)])
        dst = jnp.concatenate([ei[1], jnp.full((npad_e,), N, jnp.int32)])
        return (src.reshape(NT, NBLK, BLK, K), dst.reshape(NT, NBLK, BLK, K))
    src_s4, dst_s4 = prep(edge_index_s)
    src_t4, dst_t4 = prep(edge_index_t)
    src4 = jnp.stack([src_s4, src_t4])
    dst4 = jnp.stack([dst_s4, dst_t4])

    deg2 = _deg_kernel(dst4, jnp.zeros((NPAD,), f32))
    deg3 = jnp.concatenate(
        [deg2[:, :N], jnp.zeros((2, NP2 - N), f32)], axis=1)[..., None]

    def p(name):
        return params[name]

    def row(v):
        return v.reshape(1, -1)

    h = x
    for l in range(NL):
        ws = (p("conv%d_W1" % l), row(p("conv%d_b1" % l)),
              p("conv%d_W2" % l), row(p("conv%d_b2" % l)),
              p("conv%d_W3" % l), row(p("conv%d_b3" % l)), deg3)
        if l == 0:
            a, init = _conv0_call(h, *ws)
        else:
            a, init = _convbn_call(h, row(p("conv%d_gamma" % (l - 1))),
                                   row(p("conv%d_beta" % (l - 1))), *ws)
        h = _scatter_kernel(a, init, src4, dst4)

    out = pl.pallas_call(
        _post_body,
        out_shape=jax.ShapeDtypeStruct((G, 1), jnp.float32),
    )(h, row(p("conv%d_gamma" % (NL - 1))), row(p("conv%d_beta" % (NL - 1))),
      batch,
      p("gate_W1"), row(p("gate_b1")), p("gate_W2"), row(p("gate_b2")),
      p("out_W0"), row(p("out_b0")), p("out_W1"), row(p("out_b1")),
      p("out_W2"), row(p("out_b2")), p("out_W3"), row(p("out_b3")))
    return out
